# baseline (speedup 1.0000x reference)
import functools

import jax
import jax.numpy as jnp
from jax import lax
from jax.experimental import pallas as pl
from jax.experimental.pallas import tpu as pltpu
from jax.experimental.pallas import tpu_sc as plsc

N = 2048
D_IN = 1024
D_OUT = 1024
H = 2048
E = 8
TOP_K = 2

MB = 256
PADN = N * TOP_K + E * MB
NB = PADN // MB

NC = 2
NS = 16
NW = NC * NS
L = 16


def _router_kernel(x_ref, wg_ref, bg_ref,
                   pos0_ref, pos1_ref, g0_ref, g1_ref, meta_ref):
    x = x_ref[...]
    lg = jnp.dot(x, wg_ref[...], preferred_element_type=jnp.float32) + bg_ref[...]
    ii = lax.broadcasted_iota(jnp.int32, (N, E), 1)

    m1 = jnp.max(lg, axis=1, keepdims=True)
    i1 = jnp.min(jnp.where(lg == m1, ii, E), axis=1, keepdims=True)
    lg2 = jnp.where(ii == i1, -jnp.inf, lg)
    m2 = jnp.max(lg2, axis=1, keepdims=True)
    i2 = jnp.min(jnp.where(lg2 == m2, ii, E), axis=1, keepdims=True)
    p1 = 1.0 / (1.0 + jnp.exp(m2 - m1))
    p2 = 1.0 - p1

    oh0 = (ii == i1).astype(jnp.float32)
    oh1 = (ii == i2).astype(jnp.float32)
    rr = lax.broadcasted_iota(jnp.int32, (N, N), 0)
    cc = lax.broadcasted_iota(jnp.int32, (N, N), 1)
    tri = (cc <= rr).astype(jnp.float32)
    cb = jnp.dot(tri, jnp.concatenate([oh0, oh1], axis=1),
                 preferred_element_type=jnp.float32)
    c0, c1 = cb[:, :E], cb[:, E:]

    tot0 = c0[N - 1:N, :]
    tot1 = c1[N - 1:N, :]
    rank0 = jnp.sum(jnp.where(ii == i1, c0, 0.0), axis=1, keepdims=True) - 1.0
    rank1 = jnp.sum(jnp.where(ii == i2, c1, 0.0), axis=1, keepdims=True) - 1.0

    tot = (tot0 + tot1).astype(jnp.int32)
    padded = ((tot + MB - 1) // MB) * MB
    padded_f = padded.astype(jnp.float32)

    er = lax.broadcasted_iota(jnp.int32, (E, E), 0)
    ec = lax.broadcasted_iota(jnp.int32, (E, E), 1)
    excl = (er < ec).astype(jnp.float32)
    offs = jnp.dot(padded_f, excl, preferred_element_type=jnp.float32)

    sel0 = jnp.sum(jnp.where(ii == i1, offs, 0.0), axis=1, keepdims=True)
    sel1 = jnp.sum(jnp.where(ii == i2, offs + tot0, 0.0), axis=1, keepdims=True)
    pos0_ref[...] = (sel0 + rank0).astype(jnp.int32)
    pos1_ref[...] = (sel1 + rank1).astype(jnp.int32)
    g0_ref[...] = p1
    g1_ref[...] = p2

    cpi = (offs + padded_f).astype(jnp.int32)
    nvb = cpi[:, E - 1:E] // MB
    e8 = lax.broadcasted_iota(jnp.int32, (1, E), 1)
    last_e = jnp.max(jnp.where(padded > 0, e8, 0), axis=1, keepdims=True)
    extra = jnp.where(e8 == 0, nvb, jnp.where(e8 == 1, last_e, 0))
    meta_ref[...] = jnp.concatenate([cpi, extra], axis=1)


def _router(x, Wg, bg):
    return pl.pallas_call(
        _router_kernel,
        grid=(1,),
        in_specs=[
            pl.BlockSpec((N, D_IN), lambda i: (0, 0)),
            pl.BlockSpec((D_IN, E), lambda i: (0, 0)),
            pl.BlockSpec((1, E), lambda i: (0, 0)),
        ],
        out_specs=[
            pl.BlockSpec((N, 1), lambda i: (0, 0)),
            pl.BlockSpec((N, 1), lambda i: (0, 0)),
            pl.BlockSpec((N, 1), lambda i: (0, 0)),
            pl.BlockSpec((N, 1), lambda i: (0, 0)),
            pl.BlockSpec((1, 16), lambda i: (0, 0)),
        ],
        out_shape=[
            jax.ShapeDtypeStruct((N, 1), jnp.int32),
            jax.ShapeDtypeStruct((N, 1), jnp.int32),
            jax.ShapeDtypeStruct((N, 1), jnp.float32),
            jax.ShapeDtypeStruct((N, 1), jnp.float32),
            jax.ShapeDtypeStruct((1, 16), jnp.int32),
        ],
    )(x, Wg, bg[None, :])


def _sc_mesh():
    return plsc.VectorSubcoreMesh(core_axis_name="c", subcore_axis_name="s",
                                  num_cores=NC, num_subcores=NS)


@functools.cache
def _make_sc_scatter():
    return functools.partial(
        pl.kernel,
        out_type=[jax.ShapeDtypeStruct((PADN,), jnp.int32),
                  jax.ShapeDtypeStruct((PADN,), jnp.float32)],
        mesh=_sc_mesh(),
        scratch_types=[
            pltpu.VMEM((N,), jnp.int32),
            pltpu.VMEM((N,), jnp.int32),
            pltpu.VMEM((N,), jnp.float32),
            pltpu.VMEM((N,), jnp.float32),
            pltpu.VMEM((PADN,), jnp.int32),
            pltpu.VMEM((PADN,), jnp.float32),
        ],
        compiler_params=pltpu.CompilerParams(needs_layout_passes=False),
    )(_sc_scatter_body)


def _sc_scatter_body(pos0_hbm, pos1_hbm, g0_hbm, g1_hbm, tid_hbm, gate_hbm,
                     pos0_v, pos1_v, g0_v, g1_v, tid_v, gate_v):
    wid = lax.axis_index("s") * NC + lax.axis_index("c")

    @pl.when(wid == 0)
    def _():
        pltpu.sync_copy(pos0_hbm, pos0_v)
        pltpu.sync_copy(pos1_hbm, pos1_v)
        pltpu.sync_copy(g0_hbm, g0_v)
        pltpu.sync_copy(g1_hbm, g1_v)

        def zero(i, _):
            tid_v[pl.ds(i * L, L)] = jnp.zeros((L,), jnp.int32)
            gate_v[pl.ds(i * L, L)] = jnp.zeros((L,), jnp.float32)
            return 0
        lax.fori_loop(0, PADN // L, zero, 0)

        def scat(c, _):
            base = c * L
            tvals = lax.broadcasted_iota(jnp.int32, (L,), 0) + base
            pv0 = pos0_v[pl.ds(base, L)]
            plsc.store_scatter(tid_v, [pv0], tvals)
            plsc.store_scatter(gate_v, [pv0], g0_v[pl.ds(base, L)])
            pv1 = pos1_v[pl.ds(base, L)]
            plsc.store_scatter(tid_v, [pv1], tvals)
            plsc.store_scatter(gate_v, [pv1], g1_v[pl.ds(base, L)])
            return 0
        lax.fori_loop(0, N // L, scat, 0)

        pltpu.sync_copy(tid_v, tid_hbm)
        pltpu.sync_copy(gate_v, gate_hbm)


_GCHUNK = 24
_GRING = 4
_ROWS_PER_W = PADN // NW
_GNCH = _ROWS_PER_W // _GCHUNK


@functools.cache
def _make_sc_gather():
    return functools.partial(
        pl.kernel,
        out_type=jax.ShapeDtypeStruct((PADN, D_IN), jnp.float32),
        mesh=_sc_mesh(),
        scratch_types=(
            [pltpu.VMEM((_ROWS_PER_W,), jnp.int32)]
            + [pltpu.VMEM((_GCHUNK, D_IN), jnp.float32)] * _GRING
            + [pltpu.SemaphoreType.DMA] * (2 * _GRING)
        ),
    )(_sc_gather_body)


def _sc_gather_body(tid_hbm, x_hbm, xs_hbm, idx_v, *rest):
    bufs = rest[:_GRING]
    gsems = rest[_GRING:2 * _GRING]
    ssems = rest[2 * _GRING:]
    wid = lax.axis_index("s") * NC + lax.axis_index("c")
    base = wid * _ROWS_PER_W
    pltpu.sync_copy(tid_hbm.at[pl.ds(base, _ROWS_PER_W)], idx_v)
    gd = [None] * _GNCH
    sd = [None] * _GNCH

    def start_gather(c):
        b = c % _GRING
        if c - _GRING >= 0:
            sd[c - _GRING].wait()
        gd[c] = pltpu.async_copy(
            x_hbm.at[idx_v.at[pl.ds(c * _GCHUNK, _GCHUNK)]],
            bufs[b], gsems[b])

    for c in range(min(_GRING, _GNCH)):
        start_gather(c)
    for c in range(_GNCH):
        b = c % _GRING
        gd[c].wait()
        sd[c] = pltpu.async_copy(
            bufs[b], xs_hbm.at[pl.ds(base + c * _GCHUNK, _GCHUNK)], ssems[b])
        if c + _GRING < _GNCH:
            start_gather(c + _GRING)
    for c in range(max(0, _GNCH - _GRING), _GNCH):
        sd[c].wait()


def _mlp_kernel(m_ref, xs_ref, w1_ref, b1_ref, w2_ref, b2_ref, g_ref, out_ref):
    b = pl.program_id(0)

    @pl.when(b < m_ref[0, E])
    def _():
        h = jnp.maximum(
            jnp.dot(xs_ref[...], w1_ref[0],
                    preferred_element_type=jnp.float32) + b1_ref[0], 0.0)
        y = jnp.dot(h, w2_ref[0], preferred_element_type=jnp.float32) + b2_ref[0]
        out_ref[...] = y * g_ref[...]


def _block_expert(b, m):
    e = sum((b * MB >= m[0, j]).astype(jnp.int32) for j in range(E))
    return jnp.minimum(e, m[0, E + 1])


def _grouped_mlp(meta, xs, W1, b1, W2, b2, gate_sorted):
    grid_spec = pltpu.PrefetchScalarGridSpec(
        num_scalar_prefetch=1,
        grid=(NB,),
        in_specs=[
            pl.BlockSpec((MB, D_IN),
                         lambda b, m: (jnp.minimum(b, m[0, E] - 1), 0)),
            pl.BlockSpec((1, D_IN, H), lambda b, m: (_block_expert(b, m), 0, 0)),
            pl.BlockSpec((1, 1, H), lambda b, m: (_block_expert(b, m), 0, 0)),
            pl.BlockSpec((1, H, D_OUT), lambda b, m: (_block_expert(b, m), 0, 0)),
            pl.BlockSpec((1, 1, D_OUT), lambda b, m: (_block_expert(b, m), 0, 0)),
            pl.BlockSpec((MB, 1),
                         lambda b, m: (jnp.minimum(b, m[0, E] - 1), 0)),
        ],
        out_specs=pl.BlockSpec((MB, D_OUT), lambda b, m: (b, 0)),
    )
    return pl.pallas_call(
        _mlp_kernel,
        grid_spec=grid_spec,
        out_shape=jax.ShapeDtypeStruct((PADN, D_OUT), jnp.float32),
        compiler_params=pltpu.CompilerParams(
            dimension_semantics=("arbitrary",)),
    )(meta, xs, W1, b1[:, None, :], W2, b2[:, None, :], gate_sorted)


_TCHUNK = 16
_TOK_PER_W = N // NW
_TNCH = _TOK_PER_W // _TCHUNK


@functools.cache
def _make_sc_combine():
    return functools.partial(
        pl.kernel,
        out_type=jax.ShapeDtypeStruct((N, D_OUT), jnp.float32),
        mesh=_sc_mesh(),
        scratch_types=[
            pltpu.VMEM((_TOK_PER_W,), jnp.int32),
            pltpu.VMEM((_TOK_PER_W,), jnp.int32),
            pltpu.VMEM((_TCHUNK, D_OUT), jnp.float32),
            pltpu.VMEM((_TCHUNK, D_OUT), jnp.float32),
            pltpu.VMEM((_TCHUNK, D_OUT), jnp.float32),
            pltpu.VMEM((_TCHUNK, D_OUT), jnp.float32),
            pltpu.SemaphoreType.DMA,
            pltpu.SemaphoreType.DMA,
            pltpu.SemaphoreType.DMA,
            pltpu.SemaphoreType.DMA,
        ],
    )(_sc_combine_body)


def _sc_combine_body(pos0_hbm, pos1_hbm, ys_hbm, out_hbm,
                     idx0_v, idx1_v, a0, b0, a1, b1, g0, g1, s0, s1):
    wid = lax.axis_index("s") * NC + lax.axis_index("c")
    base = wid * _TOK_PER_W
    pltpu.sync_copy(pos0_hbm.at[pl.ds(base, _TOK_PER_W)], idx0_v)
    pltpu.sync_copy(pos1_hbm.at[pl.ds(base, _TOK_PER_W)], idx1_v)
    abufs, bbufs = (a0, a1), (b0, b1)
    gsems, ssems = (g0, g1), (s0, s1)
    gda = [None] * _TNCH
    gdb = [None] * _TNCH
    sd = [None] * _TNCH

    def start_gathers(c):
        r = c % 2
        if c - 2 >= 0:
            sd[c - 2].wait()
        gda[c] = pltpu.async_copy(
            ys_hbm.at[idx0_v.at[pl.ds(c * _TCHUNK, _TCHUNK)]],
            abufs[r], gsems[r])
        gdb[c] = pltpu.async_copy(
            ys_hbm.at[idx1_v.at[pl.ds(c * _TCHUNK, _TCHUNK)]],
            bbufs[r], gsems[r])

    start_gathers(0)
    if _TNCH > 1:
        start_gathers(1)
    for c in range(_TNCH):
        r = c % 2
        gda[c].wait()
        gdb[c].wait()
        av, bv = abufs[r], bbufs[r]

        @plsc.parallel_loop(0, _TCHUNK * (D_OUT // L), unroll=8)
        def _(q):
            j = q >> 6
            col = (q & 63) * L
            av[j, pl.ds(col, L)] += bv[j, pl.ds(col, L)]

        sd[c] = pltpu.async_copy(
            av, out_hbm.at[pl.ds(base + c * _TCHUNK, _TCHUNK)], ssems[r])
        if c + 2 < _TNCH:
            start_gathers(c + 2)
    for c in range(max(0, _TNCH - 2), _TNCH):
        sd[c].wait()


@jax.jit
def kernel(x, Wg, bg, W1, b1, W2, b2):
    pos0, pos1, g0, g1, meta = _router(x, Wg, bg)
    pos0f, pos1f = pos0.reshape(N), pos1.reshape(N)
    tid_sorted, gate_sorted = _make_sc_scatter()(pos0f, pos1f,
                                                 g0.reshape(N), g1.reshape(N))
    xs = _make_sc_gather()(tid_sorted, x)
    ys = _grouped_mlp(meta, xs, W1, b1, W2, b2, gate_sorted.reshape(PADN, 1))
    return _make_sc_combine()(pos0f, pos1f, ys)


# speedup vs baseline: 1.5994x; 1.5994x over previous
import functools

import jax
import jax.numpy as jnp
from jax import lax
from jax.experimental import pallas as pl
from jax.experimental.pallas import tpu as pltpu
from jax.experimental.pallas import tpu_sc as plsc

N = 2048
D_IN = 1024
D_OUT = 1024
H = 2048
E = 8
TOP_K = 2

MB = 256
PADN = N * TOP_K + E * MB
NB = PADN // MB

NC = 2
NS = 16
NW = NC * NS
L = 16


def _router_kernel(x_ref, wg_ref, bg_ref,
                   pos0_ref, pos1_ref, g0_ref, g1_ref, meta_ref):
    x = x_ref[...]
    lg = jnp.dot(x, wg_ref[...], preferred_element_type=jnp.float32) + bg_ref[...]
    ii = lax.broadcasted_iota(jnp.int32, (N, E), 1)

    m1 = jnp.max(lg, axis=1, keepdims=True)
    i1 = jnp.min(jnp.where(lg == m1, ii, E), axis=1, keepdims=True)
    lg2 = jnp.where(ii == i1, -jnp.inf, lg)
    m2 = jnp.max(lg2, axis=1, keepdims=True)
    i2 = jnp.min(jnp.where(lg2 == m2, ii, E), axis=1, keepdims=True)
    p1 = 1.0 / (1.0 + jnp.exp(m2 - m1))
    p2 = 1.0 - p1

    oh0 = (ii == i1).astype(jnp.float32)
    oh1 = (ii == i2).astype(jnp.float32)
    rr = lax.broadcasted_iota(jnp.int32, (N, N), 0)
    cc = lax.broadcasted_iota(jnp.int32, (N, N), 1)
    tri = (cc <= rr).astype(jnp.float32)
    cb = jnp.dot(tri, jnp.concatenate([oh0, oh1], axis=1),
                 preferred_element_type=jnp.float32)
    c0, c1 = cb[:, :E], cb[:, E:]

    tot0 = c0[N - 1:N, :]
    tot1 = c1[N - 1:N, :]
    rank0 = jnp.sum(jnp.where(ii == i1, c0, 0.0), axis=1, keepdims=True) - 1.0
    rank1 = jnp.sum(jnp.where(ii == i2, c1, 0.0), axis=1, keepdims=True) - 1.0

    tot = (tot0 + tot1).astype(jnp.int32)
    padded = ((tot + MB - 1) // MB) * MB
    padded_f = padded.astype(jnp.float32)

    er = lax.broadcasted_iota(jnp.int32, (E, E), 0)
    ec = lax.broadcasted_iota(jnp.int32, (E, E), 1)
    excl = (er < ec).astype(jnp.float32)
    offs = jnp.dot(padded_f, excl, preferred_element_type=jnp.float32)

    sel0 = jnp.sum(jnp.where(ii == i1, offs, 0.0), axis=1, keepdims=True)
    sel1 = jnp.sum(jnp.where(ii == i2, offs + tot0, 0.0), axis=1, keepdims=True)
    pos0_ref[...] = (sel0 + rank0).astype(jnp.int32)
    pos1_ref[...] = (sel1 + rank1).astype(jnp.int32)
    g0_ref[...] = p1
    g1_ref[...] = p2

    cpi = (offs + padded_f).astype(jnp.int32)
    nvb = cpi[:, E - 1:E] // MB
    e8 = lax.broadcasted_iota(jnp.int32, (1, E), 1)
    last_e = jnp.max(jnp.where(padded > 0, e8, 0), axis=1, keepdims=True)
    extra = jnp.where(e8 == 0, nvb, jnp.where(e8 == 1, last_e, 0))
    meta_ref[...] = jnp.concatenate([cpi, extra], axis=1)


def _router(x, Wg, bg):
    return pl.pallas_call(
        _router_kernel,
        grid=(1,),
        in_specs=[
            pl.BlockSpec((N, D_IN), lambda i: (0, 0)),
            pl.BlockSpec((D_IN, E), lambda i: (0, 0)),
            pl.BlockSpec((1, E), lambda i: (0, 0)),
        ],
        out_specs=[
            pl.BlockSpec((N, 1), lambda i: (0, 0)),
            pl.BlockSpec((N, 1), lambda i: (0, 0)),
            pl.BlockSpec((N, 1), lambda i: (0, 0)),
            pl.BlockSpec((N, 1), lambda i: (0, 0)),
            pl.BlockSpec((1, 16), lambda i: (0, 0)),
        ],
        out_shape=[
            jax.ShapeDtypeStruct((N, 1), jnp.int32),
            jax.ShapeDtypeStruct((N, 1), jnp.int32),
            jax.ShapeDtypeStruct((N, 1), jnp.float32),
            jax.ShapeDtypeStruct((N, 1), jnp.float32),
            jax.ShapeDtypeStruct((1, 16), jnp.int32),
        ],
    )(x, Wg, bg[None, :])


def _sc_mesh():
    return plsc.VectorSubcoreMesh(core_axis_name="c", subcore_axis_name="s",
                                  num_cores=NC, num_subcores=NS)


@functools.cache
def _make_sc_scatter():
    return functools.partial(
        pl.kernel,
        out_type=[jax.ShapeDtypeStruct((PADN,), jnp.int32),
                  jax.ShapeDtypeStruct((PADN,), jnp.float32)],
        mesh=_sc_mesh(),
        scratch_types=[
            pltpu.VMEM((N,), jnp.int32),
            pltpu.VMEM((N,), jnp.int32),
            pltpu.VMEM((N,), jnp.float32),
            pltpu.VMEM((N,), jnp.float32),
            pltpu.VMEM((PADN,), jnp.int32),
            pltpu.VMEM((PADN,), jnp.float32),
        ],
        compiler_params=pltpu.CompilerParams(needs_layout_passes=False),
    )(_sc_scatter_body)


def _sc_scatter_body(pos0_hbm, pos1_hbm, g0_hbm, g1_hbm, tid_hbm, gate_hbm,
                     pos0_v, pos1_v, g0_v, g1_v, tid_v, gate_v):
    wid = lax.axis_index("s") * NC + lax.axis_index("c")

    @pl.when(wid == 0)
    def _():
        pltpu.sync_copy(pos0_hbm, pos0_v)
        pltpu.sync_copy(pos1_hbm, pos1_v)
        pltpu.sync_copy(g0_hbm, g0_v)
        pltpu.sync_copy(g1_hbm, g1_v)

        def zero(i, _):
            fill = (lax.broadcasted_iota(jnp.int32, (L,), 0) + i * L) & (N - 1)
            tid_v[pl.ds(i * L, L)] = fill
            gate_v[pl.ds(i * L, L)] = jnp.zeros((L,), jnp.float32)
            return 0
        lax.fori_loop(0, PADN // L, zero, 0)

        def scat(c, _):
            base = c * L
            tvals = lax.broadcasted_iota(jnp.int32, (L,), 0) + base
            pv0 = pos0_v[pl.ds(base, L)]
            plsc.store_scatter(tid_v, [pv0], tvals)
            plsc.store_scatter(gate_v, [pv0], g0_v[pl.ds(base, L)])
            pv1 = pos1_v[pl.ds(base, L)]
            plsc.store_scatter(tid_v, [pv1], tvals)
            plsc.store_scatter(gate_v, [pv1], g1_v[pl.ds(base, L)])
            return 0
        lax.fori_loop(0, N // L, scat, 0)

        pltpu.sync_copy(tid_v, tid_hbm)
        pltpu.sync_copy(gate_v, gate_hbm)


_GCHUNK = 24
_GRING = 4
_ROWS_PER_W = PADN // NW
_GNCH = _ROWS_PER_W // _GCHUNK


@functools.cache
def _make_sc_gather():
    return functools.partial(
        pl.kernel,
        out_type=jax.ShapeDtypeStruct((PADN, D_IN), jnp.float32),
        mesh=_sc_mesh(),
        scratch_types=(
            [pltpu.VMEM((_ROWS_PER_W,), jnp.int32)]
            + [pltpu.VMEM((_GCHUNK, D_IN), jnp.float32)] * _GRING
            + [pltpu.SemaphoreType.DMA] * (2 * _GRING)
        ),
    )(_sc_gather_body)


def _sc_gather_body(tid_hbm, x_hbm, xs_hbm, idx_v, *rest):
    bufs = rest[:_GRING]
    gsems = rest[_GRING:2 * _GRING]
    ssems = rest[2 * _GRING:]
    wid = lax.axis_index("s") * NC + lax.axis_index("c")
    base = wid * _ROWS_PER_W
    pltpu.sync_copy(tid_hbm.at[pl.ds(base, _ROWS_PER_W)], idx_v)
    gd = [None] * _GNCH
    sd = [None] * _GNCH

    def start_gather(c):
        b = c % _GRING
        if c - _GRING >= 0:
            sd[c - _GRING].wait()
        gd[c] = pltpu.async_copy(
            x_hbm.at[idx_v.at[pl.ds(c * _GCHUNK, _GCHUNK)]],
            bufs[b], gsems[b])

    for c in range(min(_GRING, _GNCH)):
        start_gather(c)
    for c in range(_GNCH):
        b = c % _GRING
        gd[c].wait()
        sd[c] = pltpu.async_copy(
            bufs[b], xs_hbm.at[pl.ds(base + c * _GCHUNK, _GCHUNK)], ssems[b])
        if c + _GRING < _GNCH:
            start_gather(c + _GRING)
    for c in range(max(0, _GNCH - _GRING), _GNCH):
        sd[c].wait()


def _mlp_kernel(m_ref, xs_ref, w1_ref, b1_ref, w2_ref, b2_ref, g_ref, out_ref):
    b = pl.program_id(0)

    @pl.when(b < m_ref[0, E])
    def _():
        h = jnp.maximum(
            jnp.dot(xs_ref[...], w1_ref[0],
                    preferred_element_type=jnp.float32) + b1_ref[0], 0.0)
        y = jnp.dot(h, w2_ref[0], preferred_element_type=jnp.float32) + b2_ref[0]
        out_ref[...] = y * g_ref[...]


def _block_expert(b, m):
    e = sum((b * MB >= m[0, j]).astype(jnp.int32) for j in range(E))
    return jnp.minimum(e, m[0, E + 1])


def _grouped_mlp(meta, xs, W1, b1, W2, b2, gate_sorted):
    grid_spec = pltpu.PrefetchScalarGridSpec(
        num_scalar_prefetch=1,
        grid=(NB,),
        in_specs=[
            pl.BlockSpec((MB, D_IN),
                         lambda b, m: (jnp.minimum(b, m[0, E] - 1), 0)),
            pl.BlockSpec((1, D_IN, H), lambda b, m: (_block_expert(b, m), 0, 0)),
            pl.BlockSpec((1, 1, H), lambda b, m: (_block_expert(b, m), 0, 0)),
            pl.BlockSpec((1, H, D_OUT), lambda b, m: (_block_expert(b, m), 0, 0)),
            pl.BlockSpec((1, 1, D_OUT), lambda b, m: (_block_expert(b, m), 0, 0)),
            pl.BlockSpec((MB, 1),
                         lambda b, m: (jnp.minimum(b, m[0, E] - 1), 0)),
        ],
        out_specs=pl.BlockSpec((MB, D_OUT), lambda b, m: (b, 0)),
    )
    return pl.pallas_call(
        _mlp_kernel,
        grid_spec=grid_spec,
        out_shape=jax.ShapeDtypeStruct((PADN, D_OUT), jnp.float32),
        compiler_params=pltpu.CompilerParams(
            dimension_semantics=("arbitrary",)),
    )(meta, xs, W1, b1[:, None, :], W2, b2[:, None, :], gate_sorted)


_TCHUNK = 16
_TOK_PER_W = N // NW
_TNCH = _TOK_PER_W // _TCHUNK


@functools.cache
def _make_sc_combine():
    return functools.partial(
        pl.kernel,
        out_type=jax.ShapeDtypeStruct((N, D_OUT), jnp.float32),
        mesh=_sc_mesh(),
        scratch_types=[
            pltpu.VMEM((_TOK_PER_W,), jnp.int32),
            pltpu.VMEM((_TOK_PER_W,), jnp.int32),
            pltpu.VMEM((_TCHUNK, D_OUT), jnp.float32),
            pltpu.VMEM((_TCHUNK, D_OUT), jnp.float32),
            pltpu.VMEM((_TCHUNK, D_OUT), jnp.float32),
            pltpu.VMEM((_TCHUNK, D_OUT), jnp.float32),
            pltpu.SemaphoreType.DMA,
            pltpu.SemaphoreType.DMA,
            pltpu.SemaphoreType.DMA,
            pltpu.SemaphoreType.DMA,
        ],
    )(_sc_combine_body)


def _sc_combine_body(pos0_hbm, pos1_hbm, ys_hbm, out_hbm,
                     idx0_v, idx1_v, a0, b0, a1, b1, g0, g1, s0, s1):
    wid = lax.axis_index("s") * NC + lax.axis_index("c")
    base = wid * _TOK_PER_W
    pltpu.sync_copy(pos0_hbm.at[pl.ds(base, _TOK_PER_W)], idx0_v)
    pltpu.sync_copy(pos1_hbm.at[pl.ds(base, _TOK_PER_W)], idx1_v)
    abufs, bbufs = (a0, a1), (b0, b1)
    gsems, ssems = (g0, g1), (s0, s1)
    gda = [None] * _TNCH
    gdb = [None] * _TNCH
    sd = [None] * _TNCH

    def start_gathers(c):
        r = c % 2
        if c - 2 >= 0:
            sd[c - 2].wait()
        gda[c] = pltpu.async_copy(
            ys_hbm.at[idx0_v.at[pl.ds(c * _TCHUNK, _TCHUNK)]],
            abufs[r], gsems[r])
        gdb[c] = pltpu.async_copy(
            ys_hbm.at[idx1_v.at[pl.ds(c * _TCHUNK, _TCHUNK)]],
            bbufs[r], gsems[r])

    start_gathers(0)
    if _TNCH > 1:
        start_gathers(1)
    for c in range(_TNCH):
        r = c % 2
        gda[c].wait()
        gdb[c].wait()
        av, bv = abufs[r], bbufs[r]

        @plsc.parallel_loop(0, _TCHUNK * (D_OUT // L), unroll=8)
        def _(q):
            j = q >> 6
            col = (q & 63) * L
            av[j, pl.ds(col, L)] += bv[j, pl.ds(col, L)]

        sd[c] = pltpu.async_copy(
            av, out_hbm.at[pl.ds(base + c * _TCHUNK, _TCHUNK)], ssems[r])
        if c + 2 < _TNCH:
            start_gathers(c + 2)
    for c in range(max(0, _TNCH - 2), _TNCH):
        sd[c].wait()


@jax.jit
def kernel(x, Wg, bg, W1, b1, W2, b2):
    pos0, pos1, g0, g1, meta = _router(x, Wg, bg)
    pos0f, pos1f = pos0.reshape(N), pos1.reshape(N)
    tid_sorted, gate_sorted = _make_sc_scatter()(pos0f, pos1f,
                                                 g0.reshape(N), g1.reshape(N))
    xs = _make_sc_gather()(tid_sorted, x)
    ys = _grouped_mlp(meta, xs, W1, b1, W2, b2, gate_sorted.reshape(PADN, 1))
    return _make_sc_combine()(pos0f, pos1f, ys)


# speedup vs baseline: 1.6047x; 1.0033x over previous
import functools

import jax
import jax.numpy as jnp
from jax import lax
from jax.experimental import pallas as pl
from jax.experimental.pallas import tpu as pltpu
from jax.experimental.pallas import tpu_sc as plsc

N = 2048
D_IN = 1024
D_OUT = 1024
H = 2048
E = 8
TOP_K = 2

MB = 256
PADN = N * TOP_K + E * MB
NB = PADN // MB

NC = 2
NS = 16
NW = NC * NS
L = 16


def _router_kernel(x_ref, wg_ref, bg_ref,
                   pos0_ref, pos1_ref, g0_ref, g1_ref, meta_ref):
    x = x_ref[...]
    lg = jnp.dot(x, wg_ref[...], preferred_element_type=jnp.float32) + bg_ref[...]
    ii = lax.broadcasted_iota(jnp.int32, (N, E), 1)

    m1 = jnp.max(lg, axis=1, keepdims=True)
    i1 = jnp.min(jnp.where(lg == m1, ii, E), axis=1, keepdims=True)
    lg2 = jnp.where(ii == i1, -jnp.inf, lg)
    m2 = jnp.max(lg2, axis=1, keepdims=True)
    i2 = jnp.min(jnp.where(lg2 == m2, ii, E), axis=1, keepdims=True)
    p1 = 1.0 / (1.0 + jnp.exp(m2 - m1))
    p2 = 1.0 - p1

    oh0 = (ii == i1).astype(jnp.float32)
    oh1 = (ii == i2).astype(jnp.float32)
    rr = lax.broadcasted_iota(jnp.int32, (N, N), 0)
    cc = lax.broadcasted_iota(jnp.int32, (N, N), 1)
    tri = (cc <= rr).astype(jnp.float32)
    cb = jnp.dot(tri, jnp.concatenate([oh0, oh1], axis=1),
                 preferred_element_type=jnp.float32)
    c0, c1 = cb[:, :E], cb[:, E:]

    tot0 = c0[N - 1:N, :]
    tot1 = c1[N - 1:N, :]
    rank0 = jnp.sum(jnp.where(ii == i1, c0, 0.0), axis=1, keepdims=True) - 1.0
    rank1 = jnp.sum(jnp.where(ii == i2, c1, 0.0), axis=1, keepdims=True) - 1.0

    tot = (tot0 + tot1).astype(jnp.int32)
    padded = ((tot + MB - 1) // MB) * MB
    padded_f = padded.astype(jnp.float32)

    er = lax.broadcasted_iota(jnp.int32, (E, E), 0)
    ec = lax.broadcasted_iota(jnp.int32, (E, E), 1)
    excl = (er < ec).astype(jnp.float32)
    offs = jnp.dot(padded_f, excl, preferred_element_type=jnp.float32)

    sel0 = jnp.sum(jnp.where(ii == i1, offs, 0.0), axis=1, keepdims=True)
    sel1 = jnp.sum(jnp.where(ii == i2, offs + tot0, 0.0), axis=1, keepdims=True)
    pos0_ref[...] = (sel0 + rank0).astype(jnp.int32)
    pos1_ref[...] = (sel1 + rank1).astype(jnp.int32)
    g0_ref[...] = p1
    g1_ref[...] = p2

    cpi = (offs + padded_f).astype(jnp.int32)
    nvb = cpi[:, E - 1:E] // MB
    e8 = lax.broadcasted_iota(jnp.int32, (1, E), 1)
    last_e = jnp.max(jnp.where(padded > 0, e8, 0), axis=1, keepdims=True)
    extra = jnp.where(e8 == 0, nvb, jnp.where(e8 == 1, last_e, 0))
    meta_ref[...] = jnp.concatenate([cpi, extra], axis=1)


def _router(x, Wg, bg):
    return pl.pallas_call(
        _router_kernel,
        grid=(1,),
        in_specs=[
            pl.BlockSpec((N, D_IN), lambda i: (0, 0)),
            pl.BlockSpec((D_IN, E), lambda i: (0, 0)),
            pl.BlockSpec((1, E), lambda i: (0, 0)),
        ],
        out_specs=[
            pl.BlockSpec((N, 1), lambda i: (0, 0)),
            pl.BlockSpec((N, 1), lambda i: (0, 0)),
            pl.BlockSpec((N, 1), lambda i: (0, 0)),
            pl.BlockSpec((N, 1), lambda i: (0, 0)),
            pl.BlockSpec((1, 16), lambda i: (0, 0)),
        ],
        out_shape=[
            jax.ShapeDtypeStruct((N, 1), jnp.int32),
            jax.ShapeDtypeStruct((N, 1), jnp.int32),
            jax.ShapeDtypeStruct((N, 1), jnp.float32),
            jax.ShapeDtypeStruct((N, 1), jnp.float32),
            jax.ShapeDtypeStruct((1, 16), jnp.int32),
        ],
    )(x, Wg, bg[None, :])


def _sc_mesh():
    return plsc.VectorSubcoreMesh(core_axis_name="c", subcore_axis_name="s",
                                  num_cores=NC, num_subcores=NS)


@functools.cache
def _make_sc_scatter():
    return functools.partial(
        pl.kernel,
        out_type=[jax.ShapeDtypeStruct((PADN,), jnp.int32),
                  jax.ShapeDtypeStruct((PADN,), jnp.float32)],
        mesh=_sc_mesh(),
        scratch_types=[
            pltpu.VMEM((N,), jnp.int32),
            pltpu.VMEM((N,), jnp.int32),
            pltpu.VMEM((N,), jnp.float32),
            pltpu.VMEM((N,), jnp.float32),
            pltpu.VMEM((PADN,), jnp.int32),
            pltpu.VMEM((PADN,), jnp.float32),
        ],
        compiler_params=pltpu.CompilerParams(needs_layout_passes=False),
    )(_sc_scatter_body)


def _sc_scatter_body(pos0_hbm, pos1_hbm, g0_hbm, g1_hbm, tid_hbm, gate_hbm,
                     pos0_v, pos1_v, g0_v, g1_v, tid_v, gate_v):
    wid = lax.axis_index("s") * NC + lax.axis_index("c")

    @pl.when(wid == 0)
    def _():
        pltpu.sync_copy(pos0_hbm, pos0_v)
        pltpu.sync_copy(pos1_hbm, pos1_v)
        pltpu.sync_copy(g0_hbm, g0_v)
        pltpu.sync_copy(g1_hbm, g1_v)

        def zero(i, _):
            fill = (lax.broadcasted_iota(jnp.int32, (L,), 0) + i * L) & (N - 1)
            tid_v[pl.ds(i * L, L)] = fill
            gate_v[pl.ds(i * L, L)] = jnp.zeros((L,), jnp.float32)
            return 0
        lax.fori_loop(0, PADN // L, zero, 0)

        def scat(c, _):
            base = c * L
            tvals = lax.broadcasted_iota(jnp.int32, (L,), 0) + base
            pv0 = pos0_v[pl.ds(base, L)]
            plsc.store_scatter(tid_v, [pv0], tvals)
            plsc.store_scatter(gate_v, [pv0], g0_v[pl.ds(base, L)])
            pv1 = pos1_v[pl.ds(base, L)]
            plsc.store_scatter(tid_v, [pv1], tvals)
            plsc.store_scatter(gate_v, [pv1], g1_v[pl.ds(base, L)])
            return 0
        lax.fori_loop(0, N // L, scat, 0)

        pltpu.sync_copy(tid_v, tid_hbm)
        pltpu.sync_copy(gate_v, gate_hbm)


_GCHUNK = 24
_GRING = 4
_ROWS_PER_W = PADN // NW
_GNCH = _ROWS_PER_W // _GCHUNK


@functools.cache
def _make_sc_gather():
    return functools.partial(
        pl.kernel,
        out_type=jax.ShapeDtypeStruct((PADN, D_IN), jnp.float32),
        mesh=_sc_mesh(),
        scratch_types=(
            [pltpu.VMEM((_ROWS_PER_W,), jnp.int32)]
            + [pltpu.VMEM((_GCHUNK, D_IN), jnp.float32)] * _GRING
            + [pltpu.SemaphoreType.DMA] * (2 * _GRING)
        ),
    )(_sc_gather_body)


def _sc_gather_body(tid_hbm, x_hbm, xs_hbm, idx_v, *rest):
    bufs = rest[:_GRING]
    gsems = rest[_GRING:2 * _GRING]
    ssems = rest[2 * _GRING:]
    wid = lax.axis_index("s") * NC + lax.axis_index("c")
    base = wid * _ROWS_PER_W
    pltpu.sync_copy(tid_hbm.at[pl.ds(base, _ROWS_PER_W)], idx_v)
    gd = [None] * _GNCH
    sd = [None] * _GNCH

    def start_gather(c):
        b = c % _GRING
        if c - _GRING >= 0:
            sd[c - _GRING].wait()
        gd[c] = pltpu.async_copy(
            x_hbm.at[idx_v.at[pl.ds(c * _GCHUNK, _GCHUNK)]],
            bufs[b], gsems[b])

    for c in range(min(_GRING, _GNCH)):
        start_gather(c)
    for c in range(_GNCH):
        b = c % _GRING
        gd[c].wait()
        sd[c] = pltpu.async_copy(
            bufs[b], xs_hbm.at[pl.ds(base + c * _GCHUNK, _GCHUNK)], ssems[b])
        if c + _GRING < _GNCH:
            start_gather(c + _GRING)
    for c in range(max(0, _GNCH - _GRING), _GNCH):
        sd[c].wait()


def _mlp_kernel(m_ref, xs_ref, w1_ref, b1_ref, w2_ref, b2_ref, g_ref, out_ref):
    b = pl.program_id(0)

    @pl.when(b < m_ref[0, E])
    def _():
        h = jnp.maximum(
            jnp.dot(xs_ref[...].astype(jnp.bfloat16),
                    w1_ref[0].astype(jnp.bfloat16),
                    preferred_element_type=jnp.float32) + b1_ref[0], 0.0)
        y = jnp.dot(h.astype(jnp.bfloat16), w2_ref[0].astype(jnp.bfloat16),
                    preferred_element_type=jnp.float32) + b2_ref[0]
        out_ref[...] = y * g_ref[...]


def _block_expert(b, m):
    e = sum((b * MB >= m[0, j]).astype(jnp.int32) for j in range(E))
    return jnp.minimum(e, m[0, E + 1])


def _grouped_mlp(meta, xs, W1, b1, W2, b2, gate_sorted):
    grid_spec = pltpu.PrefetchScalarGridSpec(
        num_scalar_prefetch=1,
        grid=(NB,),
        in_specs=[
            pl.BlockSpec((MB, D_IN),
                         lambda b, m: (jnp.minimum(b, m[0, E] - 1), 0)),
            pl.BlockSpec((1, D_IN, H), lambda b, m: (_block_expert(b, m), 0, 0)),
            pl.BlockSpec((1, 1, H), lambda b, m: (_block_expert(b, m), 0, 0)),
            pl.BlockSpec((1, H, D_OUT), lambda b, m: (_block_expert(b, m), 0, 0)),
            pl.BlockSpec((1, 1, D_OUT), lambda b, m: (_block_expert(b, m), 0, 0)),
            pl.BlockSpec((MB, 1),
                         lambda b, m: (jnp.minimum(b, m[0, E] - 1), 0)),
        ],
        out_specs=pl.BlockSpec((MB, D_OUT), lambda b, m: (b, 0)),
    )
    return pl.pallas_call(
        _mlp_kernel,
        grid_spec=grid_spec,
        out_shape=jax.ShapeDtypeStruct((PADN, D_OUT), jnp.float32),
        compiler_params=pltpu.CompilerParams(
            dimension_semantics=("arbitrary",)),
    )(meta, xs, W1, b1[:, None, :], W2, b2[:, None, :], gate_sorted)


_TCHUNK = 16
_TOK_PER_W = N // NW
_TNCH = _TOK_PER_W // _TCHUNK


@functools.cache
def _make_sc_combine():
    return functools.partial(
        pl.kernel,
        out_type=jax.ShapeDtypeStruct((N, D_OUT), jnp.float32),
        mesh=_sc_mesh(),
        scratch_types=[
            pltpu.VMEM((_TOK_PER_W,), jnp.int32),
            pltpu.VMEM((_TOK_PER_W,), jnp.int32),
            pltpu.VMEM((_TCHUNK, D_OUT), jnp.float32),
            pltpu.VMEM((_TCHUNK, D_OUT), jnp.float32),
            pltpu.VMEM((_TCHUNK, D_OUT), jnp.float32),
            pltpu.VMEM((_TCHUNK, D_OUT), jnp.float32),
            pltpu.SemaphoreType.DMA,
            pltpu.SemaphoreType.DMA,
            pltpu.SemaphoreType.DMA,
            pltpu.SemaphoreType.DMA,
        ],
    )(_sc_combine_body)


def _sc_combine_body(pos0_hbm, pos1_hbm, ys_hbm, out_hbm,
                     idx0_v, idx1_v, a0, b0, a1, b1, g0, g1, s0, s1):
    wid = lax.axis_index("s") * NC + lax.axis_index("c")
    base = wid * _TOK_PER_W
    pltpu.sync_copy(pos0_hbm.at[pl.ds(base, _TOK_PER_W)], idx0_v)
    pltpu.sync_copy(pos1_hbm.at[pl.ds(base, _TOK_PER_W)], idx1_v)
    abufs, bbufs = (a0, a1), (b0, b1)
    gsems, ssems = (g0, g1), (s0, s1)
    gda = [None] * _TNCH
    gdb = [None] * _TNCH
    sd = [None] * _TNCH

    def start_gathers(c):
        r = c % 2
        if c - 2 >= 0:
            sd[c - 2].wait()
        gda[c] = pltpu.async_copy(
            ys_hbm.at[idx0_v.at[pl.ds(c * _TCHUNK, _TCHUNK)]],
            abufs[r], gsems[r])
        gdb[c] = pltpu.async_copy(
            ys_hbm.at[idx1_v.at[pl.ds(c * _TCHUNK, _TCHUNK)]],
            bbufs[r], gsems[r])

    start_gathers(0)
    if _TNCH > 1:
        start_gathers(1)
    for c in range(_TNCH):
        r = c % 2
        gda[c].wait()
        gdb[c].wait()
        av, bv = abufs[r], bbufs[r]

        @plsc.parallel_loop(0, _TCHUNK * (D_OUT // L), unroll=8)
        def _(q):
            j = q >> 6
            col = (q & 63) * L
            av[j, pl.ds(col, L)] += bv[j, pl.ds(col, L)]

        sd[c] = pltpu.async_copy(
            av, out_hbm.at[pl.ds(base + c * _TCHUNK, _TCHUNK)], ssems[r])
        if c + 2 < _TNCH:
            start_gathers(c + 2)
    for c in range(max(0, _TNCH - 2), _TNCH):
        sd[c].wait()


@jax.jit
def kernel(x, Wg, bg, W1, b1, W2, b2):
    pos0, pos1, g0, g1, meta = _router(x, Wg, bg)
    pos0f, pos1f = pos0.reshape(N), pos1.reshape(N)
    tid_sorted, gate_sorted = _make_sc_scatter()(pos0f, pos1f,
                                                 g0.reshape(N), g1.reshape(N))
    xs = _make_sc_gather()(tid_sorted, x)
    ys = _grouped_mlp(meta, xs, W1, b1, W2, b2, gate_sorted.reshape(PADN, 1))
    return _make_sc_combine()(pos0f, pos1f, ys)


# speedup vs baseline: 1.6268x; 1.0138x over previous
import functools

import jax
import jax.numpy as jnp
from jax import lax
from jax.experimental import pallas as pl
from jax.experimental.pallas import tpu as pltpu
from jax.experimental.pallas import tpu_sc as plsc

N = 2048
D_IN = 1024
D_OUT = 1024
H = 2048
E = 8
TOP_K = 2

MB = 512
PADN = N * TOP_K + E * MB
NB = PADN // MB

NC = 2
NS = 16
NW = NC * NS
L = 16


def _router_kernel(x_ref, wg_ref, bg_ref,
                   pos0_ref, pos1_ref, g0_ref, g1_ref, meta_ref):
    x = x_ref[...]
    lg = jnp.dot(x, wg_ref[...], preferred_element_type=jnp.float32) + bg_ref[...]
    ii = lax.broadcasted_iota(jnp.int32, (N, E), 1)

    m1 = jnp.max(lg, axis=1, keepdims=True)
    i1 = jnp.min(jnp.where(lg == m1, ii, E), axis=1, keepdims=True)
    lg2 = jnp.where(ii == i1, -jnp.inf, lg)
    m2 = jnp.max(lg2, axis=1, keepdims=True)
    i2 = jnp.min(jnp.where(lg2 == m2, ii, E), axis=1, keepdims=True)
    p1 = 1.0 / (1.0 + jnp.exp(m2 - m1))
    p2 = 1.0 - p1

    oh0 = (ii == i1).astype(jnp.float32)
    oh1 = (ii == i2).astype(jnp.float32)
    rr = lax.broadcasted_iota(jnp.int32, (N, N), 0)
    cc = lax.broadcasted_iota(jnp.int32, (N, N), 1)
    tri = (cc <= rr).astype(jnp.float32)
    cb = jnp.dot(tri, jnp.concatenate([oh0, oh1], axis=1),
                 preferred_element_type=jnp.float32)
    c0, c1 = cb[:, :E], cb[:, E:]

    tot0 = c0[N - 1:N, :]
    tot1 = c1[N - 1:N, :]
    rank0 = jnp.sum(jnp.where(ii == i1, c0, 0.0), axis=1, keepdims=True) - 1.0
    rank1 = jnp.sum(jnp.where(ii == i2, c1, 0.0), axis=1, keepdims=True) - 1.0

    tot = (tot0 + tot1).astype(jnp.int32)
    padded = ((tot + MB - 1) // MB) * MB
    padded_f = padded.astype(jnp.float32)

    er = lax.broadcasted_iota(jnp.int32, (E, E), 0)
    ec = lax.broadcasted_iota(jnp.int32, (E, E), 1)
    excl = (er < ec).astype(jnp.float32)
    offs = jnp.dot(padded_f, excl, preferred_element_type=jnp.float32)

    sel0 = jnp.sum(jnp.where(ii == i1, offs, 0.0), axis=1, keepdims=True)
    sel1 = jnp.sum(jnp.where(ii == i2, offs + tot0, 0.0), axis=1, keepdims=True)
    pos0_ref[...] = (sel0 + rank0).astype(jnp.int32)
    pos1_ref[...] = (sel1 + rank1).astype(jnp.int32)
    g0_ref[...] = p1
    g1_ref[...] = p2

    cpi = (offs + padded_f).astype(jnp.int32)
    nvb = cpi[:, E - 1:E] // MB
    e8 = lax.broadcasted_iota(jnp.int32, (1, E), 1)
    last_e = jnp.max(jnp.where(padded > 0, e8, 0), axis=1, keepdims=True)
    extra = jnp.where(e8 == 0, nvb, jnp.where(e8 == 1, last_e, 0))
    meta_ref[...] = jnp.concatenate([cpi, extra], axis=1)


def _router(x, Wg, bg):
    return pl.pallas_call(
        _router_kernel,
        grid=(1,),
        in_specs=[
            pl.BlockSpec((N, D_IN), lambda i: (0, 0)),
            pl.BlockSpec((D_IN, E), lambda i: (0, 0)),
            pl.BlockSpec((1, E), lambda i: (0, 0)),
        ],
        out_specs=[
            pl.BlockSpec((N, 1), lambda i: (0, 0)),
            pl.BlockSpec((N, 1), lambda i: (0, 0)),
            pl.BlockSpec((N, 1), lambda i: (0, 0)),
            pl.BlockSpec((N, 1), lambda i: (0, 0)),
            pl.BlockSpec((1, 16), lambda i: (0, 0)),
        ],
        out_shape=[
            jax.ShapeDtypeStruct((N, 1), jnp.int32),
            jax.ShapeDtypeStruct((N, 1), jnp.int32),
            jax.ShapeDtypeStruct((N, 1), jnp.float32),
            jax.ShapeDtypeStruct((N, 1), jnp.float32),
            jax.ShapeDtypeStruct((1, 16), jnp.int32),
        ],
    )(x, Wg, bg[None, :])


def _sc_mesh():
    return plsc.VectorSubcoreMesh(core_axis_name="c", subcore_axis_name="s",
                                  num_cores=NC, num_subcores=NS)


@functools.cache
def _make_sc_scatter():
    return functools.partial(
        pl.kernel,
        out_type=[jax.ShapeDtypeStruct((PADN,), jnp.int32),
                  jax.ShapeDtypeStruct((PADN,), jnp.float32)],
        mesh=_sc_mesh(),
        scratch_types=[
            pltpu.VMEM((N,), jnp.int32),
            pltpu.VMEM((N,), jnp.int32),
            pltpu.VMEM((N,), jnp.float32),
            pltpu.VMEM((N,), jnp.float32),
            pltpu.VMEM((PADN,), jnp.int32),
            pltpu.VMEM((PADN,), jnp.float32),
        ],
        compiler_params=pltpu.CompilerParams(needs_layout_passes=False),
    )(_sc_scatter_body)


def _sc_scatter_body(pos0_hbm, pos1_hbm, g0_hbm, g1_hbm, tid_hbm, gate_hbm,
                     pos0_v, pos1_v, g0_v, g1_v, tid_v, gate_v):
    wid = lax.axis_index("s") * NC + lax.axis_index("c")

    @pl.when(wid == 0)
    def _():
        pltpu.sync_copy(pos0_hbm, pos0_v)
        pltpu.sync_copy(pos1_hbm, pos1_v)
        pltpu.sync_copy(g0_hbm, g0_v)
        pltpu.sync_copy(g1_hbm, g1_v)

        def zero(i, _):
            fill = (lax.broadcasted_iota(jnp.int32, (L,), 0) + i * L) & (N - 1)
            tid_v[pl.ds(i * L, L)] = fill
            gate_v[pl.ds(i * L, L)] = jnp.zeros((L,), jnp.float32)
            return 0
        lax.fori_loop(0, PADN // L, zero, 0)

        def scat(c, _):
            base = c * L
            tvals = lax.broadcasted_iota(jnp.int32, (L,), 0) + base
            pv0 = pos0_v[pl.ds(base, L)]
            plsc.store_scatter(tid_v, [pv0], tvals)
            plsc.store_scatter(gate_v, [pv0], g0_v[pl.ds(base, L)])
            pv1 = pos1_v[pl.ds(base, L)]
            plsc.store_scatter(tid_v, [pv1], tvals)
            plsc.store_scatter(gate_v, [pv1], g1_v[pl.ds(base, L)])
            return 0
        lax.fori_loop(0, N // L, scat, 0)

        pltpu.sync_copy(tid_v, tid_hbm)
        pltpu.sync_copy(gate_v, gate_hbm)


_GCHUNK = 16
_GRING = 4
_ROWS_PER_W = PADN // NW
_GNCH = _ROWS_PER_W // _GCHUNK


@functools.cache
def _make_sc_gather():
    return functools.partial(
        pl.kernel,
        out_type=jax.ShapeDtypeStruct((PADN, D_IN), jnp.float32),
        mesh=_sc_mesh(),
        scratch_types=(
            [pltpu.VMEM((_ROWS_PER_W,), jnp.int32)]
            + [pltpu.VMEM((_GCHUNK, D_IN), jnp.float32)] * _GRING
            + [pltpu.SemaphoreType.DMA] * (2 * _GRING)
        ),
    )(_sc_gather_body)


def _sc_gather_body(tid_hbm, x_hbm, xs_hbm, idx_v, *rest):
    bufs = rest[:_GRING]
    gsems = rest[_GRING:2 * _GRING]
    ssems = rest[2 * _GRING:]
    wid = lax.axis_index("s") * NC + lax.axis_index("c")
    base = wid * _ROWS_PER_W
    pltpu.sync_copy(tid_hbm.at[pl.ds(base, _ROWS_PER_W)], idx_v)
    gd = [None] * _GNCH
    sd = [None] * _GNCH

    def start_gather(c):
        b = c % _GRING
        if c - _GRING >= 0:
            sd[c - _GRING].wait()
        gd[c] = pltpu.async_copy(
            x_hbm.at[idx_v.at[pl.ds(c * _GCHUNK, _GCHUNK)]],
            bufs[b], gsems[b])

    for c in range(min(_GRING, _GNCH)):
        start_gather(c)
    for c in range(_GNCH):
        b = c % _GRING
        gd[c].wait()
        sd[c] = pltpu.async_copy(
            bufs[b], xs_hbm.at[pl.ds(base + c * _GCHUNK, _GCHUNK)], ssems[b])
        if c + _GRING < _GNCH:
            start_gather(c + _GRING)
    for c in range(max(0, _GNCH - _GRING), _GNCH):
        sd[c].wait()


def _mlp_kernel(m_ref, xs_ref, w1_ref, b1_ref, w2_ref, b2_ref, g_ref, out_ref):
    b = pl.program_id(0)

    @pl.when(b < m_ref[0, E])
    def _():
        h = jnp.maximum(
            jnp.dot(xs_ref[...], w1_ref[0],
                    preferred_element_type=jnp.float32) + b1_ref[0], 0.0)
        y = jnp.dot(h, w2_ref[0], preferred_element_type=jnp.float32) + b2_ref[0]
        out_ref[...] = y * g_ref[...]


def _block_expert(b, m):
    e = sum((b * MB >= m[0, j]).astype(jnp.int32) for j in range(E))
    return jnp.minimum(e, m[0, E + 1])


def _grouped_mlp(meta, xs, W1, b1, W2, b2, gate_sorted):
    grid_spec = pltpu.PrefetchScalarGridSpec(
        num_scalar_prefetch=1,
        grid=(NB,),
        in_specs=[
            pl.BlockSpec((MB, D_IN),
                         lambda b, m: (jnp.minimum(b, m[0, E] - 1), 0)),
            pl.BlockSpec((1, D_IN, H), lambda b, m: (_block_expert(b, m), 0, 0)),
            pl.BlockSpec((1, 1, H), lambda b, m: (_block_expert(b, m), 0, 0)),
            pl.BlockSpec((1, H, D_OUT), lambda b, m: (_block_expert(b, m), 0, 0)),
            pl.BlockSpec((1, 1, D_OUT), lambda b, m: (_block_expert(b, m), 0, 0)),
            pl.BlockSpec((MB, 1),
                         lambda b, m: (jnp.minimum(b, m[0, E] - 1), 0)),
        ],
        out_specs=pl.BlockSpec((MB, D_OUT), lambda b, m: (b, 0)),
    )
    return pl.pallas_call(
        _mlp_kernel,
        grid_spec=grid_spec,
        out_shape=jax.ShapeDtypeStruct((PADN, D_OUT), jnp.float32),
        compiler_params=pltpu.CompilerParams(
            dimension_semantics=("arbitrary",)),
    )(meta, xs, W1, b1[:, None, :], W2, b2[:, None, :], gate_sorted)


_TCHUNK = 16
_TOK_PER_W = N // NW
_TNCH = _TOK_PER_W // _TCHUNK


@functools.cache
def _make_sc_combine():
    return functools.partial(
        pl.kernel,
        out_type=jax.ShapeDtypeStruct((N, D_OUT), jnp.float32),
        mesh=_sc_mesh(),
        scratch_types=[
            pltpu.VMEM((_TOK_PER_W,), jnp.int32),
            pltpu.VMEM((_TOK_PER_W,), jnp.int32),
            pltpu.VMEM((_TCHUNK, D_OUT), jnp.float32),
            pltpu.VMEM((_TCHUNK, D_OUT), jnp.float32),
            pltpu.VMEM((_TCHUNK, D_OUT), jnp.float32),
            pltpu.VMEM((_TCHUNK, D_OUT), jnp.float32),
            pltpu.SemaphoreType.DMA,
            pltpu.SemaphoreType.DMA,
            pltpu.SemaphoreType.DMA,
            pltpu.SemaphoreType.DMA,
        ],
    )(_sc_combine_body)


def _sc_combine_body(pos0_hbm, pos1_hbm, ys_hbm, out_hbm,
                     idx0_v, idx1_v, a0, b0, a1, b1, g0, g1, s0, s1):
    wid = lax.axis_index("s") * NC + lax.axis_index("c")
    base = wid * _TOK_PER_W
    pltpu.sync_copy(pos0_hbm.at[pl.ds(base, _TOK_PER_W)], idx0_v)
    pltpu.sync_copy(pos1_hbm.at[pl.ds(base, _TOK_PER_W)], idx1_v)
    abufs, bbufs = (a0, a1), (b0, b1)
    gsems, ssems = (g0, g1), (s0, s1)
    gda = [None] * _TNCH
    gdb = [None] * _TNCH
    sd = [None] * _TNCH

    def start_gathers(c):
        r = c % 2
        if c - 2 >= 0:
            sd[c - 2].wait()
        gda[c] = pltpu.async_copy(
            ys_hbm.at[idx0_v.at[pl.ds(c * _TCHUNK, _TCHUNK)]],
            abufs[r], gsems[r])
        gdb[c] = pltpu.async_copy(
            ys_hbm.at[idx1_v.at[pl.ds(c * _TCHUNK, _TCHUNK)]],
            bbufs[r], gsems[r])

    start_gathers(0)
    if _TNCH > 1:
        start_gathers(1)
    for c in range(_TNCH):
        r = c % 2
        gda[c].wait()
        gdb[c].wait()
        av, bv = abufs[r], bbufs[r]

        @plsc.parallel_loop(0, _TCHUNK * (D_OUT // L), unroll=8)
        def _(q):
            j = q >> 6
            col = (q & 63) * L
            av[j, pl.ds(col, L)] += bv[j, pl.ds(col, L)]

        sd[c] = pltpu.async_copy(
            av, out_hbm.at[pl.ds(base + c * _TCHUNK, _TCHUNK)], ssems[r])
        if c + 2 < _TNCH:
            start_gathers(c + 2)
    for c in range(max(0, _TNCH - 2), _TNCH):
        sd[c].wait()


@jax.jit
def kernel(x, Wg, bg, W1, b1, W2, b2):
    pos0, pos1, g0, g1, meta = _router(x, Wg, bg)
    pos0f, pos1f = pos0.reshape(N), pos1.reshape(N)
    tid_sorted, gate_sorted = _make_sc_scatter()(pos0f, pos1f,
                                                 g0.reshape(N), g1.reshape(N))
    xs = _make_sc_gather()(tid_sorted, x)
    ys = _grouped_mlp(meta, xs, W1, b1, W2, b2, gate_sorted.reshape(PADN, 1))
    return _make_sc_combine()(pos0f, pos1f, ys)
